# baseline (speedup 1.0000x reference)
import functools

import jax
import jax.numpy as jnp
from jax import lax
from jax.experimental import pallas as pl
from jax.experimental.pallas import tpu as pltpu
from jax.experimental.pallas import tpu_sc as plsc

NENT = 100000
NR = 26
STT = 128
ABS = 128
REL = 128
RDIM = 128
B = 16384

_NC = 2
_NS = 16
_NW = _NC * _NS
_BPW = B // _NW
_CH = 128
_NCHUNK = _BPW // _CH

_NB = 256
_GRID = B // _NB


def _sc_gather_body(sidx_hbm, oidx_hbm, sidxh_hbm, oidxh_hbm, e_emb, dfrq,
                    dphi, damp, mfrq, mphi, mamp, out_s, out_o, out_sdf,
                    out_sdp, out_sda, out_smf, out_smp, out_sma, out_odf,
                    out_odp, out_oda, out_omf, out_omp, out_oma, idx_s, idx_o,
                    idx_sh, idx_oh, buf, sem):
    wid = lax.axis_index("s") * _NC + lax.axis_index("c")
    base = wid * _BPW
    pltpu.sync_copy(sidx_hbm.at[pl.ds(wid * _NCHUNK, _NCHUNK)], idx_s)
    pltpu.sync_copy(oidx_hbm.at[pl.ds(wid * _NCHUNK, _NCHUNK)], idx_o)
    pltpu.sync_copy(sidxh_hbm.at[pl.ds(wid * _NCHUNK, _NCHUNK)], idx_sh)
    pltpu.sync_copy(oidxh_hbm.at[pl.ds(wid * _NCHUNK, _NCHUNK)], idx_oh)

    jobs = (
        (idx_s, e_emb, out_s),
        (idx_o, e_emb, out_o),
        (idx_s, dfrq, out_sdf),
        (idx_s, dphi, out_sdp),
        (idx_sh, damp, out_sda),
        (idx_s, mfrq, out_smf),
        (idx_s, mphi, out_smp),
        (idx_sh, mamp, out_sma),
        (idx_o, dfrq, out_odf),
        (idx_o, dphi, out_odp),
        (idx_oh, damp, out_oda),
        (idx_o, mfrq, out_omf),
        (idx_o, mphi, out_omp),
        (idx_oh, mamp, out_oma),
    )
    for idx, table, out in jobs:
        for c in range(_NCHUNK):
            pltpu.async_copy(table.at[idx.at[c]], buf, sem).wait()
            pltpu.sync_copy(buf, out.at[pl.ds(base + c * _CH, _CH)])


def _sc_gather(s_idx, o_idx, e_emb, dfrq, dphi, damp, mfrq, mphi, mamp):
    f32 = jnp.float32
    outs = [jax.ShapeDtypeStruct((B, 128), f32) for _ in range(14)]
    mesh = plsc.VectorSubcoreMesh(core_axis_name="c", subcore_axis_name="s")
    fn = pl.kernel(
        _sc_gather_body,
        mesh=mesh,
        out_type=outs,
        scratch_types=[
            pltpu.VMEM((_NCHUNK, _CH), jnp.int32),
            pltpu.VMEM((_NCHUNK, _CH), jnp.int32),
            pltpu.VMEM((_NCHUNK, _CH), jnp.int32),
            pltpu.VMEM((_NCHUNK, _CH), jnp.int32),
            pltpu.VMEM((_CH, 128), f32),
            pltpu.SemaphoreType.DMA,
        ],
    )
    r2 = lambda a: a.reshape(B // _CH, _CH)
    return fn(r2(s_idx), r2(o_idx), r2(s_idx >> 1), r2(o_idx >> 1),
              e_emb, dfrq, dphi, damp.reshape(NENT // 2, 128), mfrq, mphi,
              mamp.reshape(NENT // 2, 128))


def _rel_body(ridx_ref, rels_ref, relo_ref, remb_ref, wrp_ref,
              r_out_ref, sr_ref, or_ref):
    f32 = jnp.float32
    t = lax.broadcasted_iota(jnp.int32, (1, REL // 2), 1).astype(f32) * (2.0 / REL)
    frq = jnp.exp(t * (-jnp.log(10000.0)))

    ridx = ridx_ref[...]
    iota_r = lax.broadcasted_iota(jnp.int32, (1, NR), 1)
    oh = (ridx == iota_r).astype(f32)
    r_out_ref[...] = jnp.dot(oh, remb_ref[...], preferred_element_type=f32)
    w_sel = jnp.dot(oh, wrp_ref[...], preferred_element_type=f32)

    for rel_ref, out_ref in ((rels_ref, sr_ref), (relo_ref, or_ref)):
        e = rel_ref[...].astype(f32)
        acc_c = jnp.zeros((_NB, REL // 2), f32)
        acc_s = jnp.zeros((_NB, REL // 2), f32)
        for j in range(NR):
            ang = e[:, j:j + 1] * frq
            wj = w_sel[:, j:j + 1]
            acc_c = acc_c + wj * jnp.cos(ang)
            acc_s = acc_s + wj * jnp.sin(ang)
        out_ref[...] = jnp.concatenate([acc_c, acc_s], axis=1)


def _rel_kernel(r_idx, rel_s, rel_o, r_emb, w_rp2):
    f32 = jnp.float32
    blk = lambda w: pl.BlockSpec((_NB, w), lambda i: (i, 0))
    whole = lambda a, b: pl.BlockSpec((a, b), lambda i: (0, 0))
    return pl.pallas_call(
        _rel_body,
        grid=(_GRID,),
        in_specs=[blk(1), blk(NR), blk(NR), whole(NR, REL), whole(NR, NR)],
        out_specs=[blk(REL), blk(REL), blk(REL)],
        out_shape=[jax.ShapeDtypeStruct((B, REL), f32)] * 3,
    )(r_idx.reshape(B, 1), rel_s, rel_o, r_emb, w_rp2)


def _tp_body(d_ref, m_ref, spar_ref, opar_ref, we_ref, s_ref, sdf_ref,
             sdp_ref, sda_ref, smf_ref, smp_ref, sma_ref, o_ref, odf_ref,
             odp_ref, oda_ref, omf_ref, omp_ref, oma_ref,
             st_ref, sp_ref, ot_ref, op_ref):
    f32 = jnp.float32
    H = ABS // 2
    d = d_ref[...].astype(f32)
    m = m_ref[...].astype(f32)
    spar = spar_ref[...] > 0
    opar = opar_ref[...] > 0

    we = we_ref[...]
    re_w = we[:H, :]
    im_w = we[H:, :]
    w_top = jnp.concatenate([re_w, -im_w], axis=1)
    w_bot = jnp.concatenate([-im_w, re_w], axis=1)
    w_full = jnp.concatenate([w_top, w_bot], axis=0)

    def t_half(scale, frq_ref, phi_ref, amp_ref, par):
        a = scale * frq_ref[...] + phi_ref[...]
        ampf = amp_ref[...]
        amp = jnp.where(par, ampf[:, H:], ampf[:, :H])
        return jnp.concatenate(
            [amp * jnp.cos(a[:, :H]), amp * jnp.sin(a[:, H:])], axis=1)

    st_ref[...] = (t_half(d, sdf_ref, sdp_ref, sda_ref, spar) +
                   t_half(m, smf_ref, smp_ref, sma_ref, spar))
    ot_ref[...] = (t_half(d, odf_ref, odp_ref, oda_ref, opar) +
                   t_half(m, omf_ref, omp_ref, oma_ref, opar))
    sp_ref[...] = jnp.dot(s_ref[...], w_full, preferred_element_type=f32)
    op_ref[...] = jnp.dot(o_ref[...], w_full, preferred_element_type=f32)


def _tp_kernel(d, m, s_par, o_par, w_e, s_rows, sdf, sdp, sda, smf, smp, sma,
               o_rows, odf, odp, oda, omf, omp, oma):
    f32 = jnp.float32
    blk = lambda w: pl.BlockSpec((_NB, w), lambda i: (i, 0))
    side = [blk(ABS)] * 6
    return pl.pallas_call(
        _tp_body,
        grid=(_GRID,),
        in_specs=([blk(1), blk(1), blk(1), blk(1),
                   pl.BlockSpec((STT, STT // 2), lambda i: (0, 0)),
                   blk(STT)] + side + [blk(STT)] + side),
        out_specs=[blk(STT)] * 4,
        out_shape=[jax.ShapeDtypeStruct((B, STT), f32)] * 4,
    )(d.reshape(B, 1), m.reshape(B, 1), s_par.reshape(B, 1),
      o_par.reshape(B, 1), w_e, s_rows, sdf, sdp, sda, smf, smp,
      sma, o_rows, odf, odp, oda, omf, omp, oma)


def kernel(x, e_emb, r_emb, abs_d_frq_emb, abs_d_phi_emb, abs_d_amp_emb,
           abs_m_frq_emb, abs_m_phi_emb, abs_m_amp_emb, w_e, w_rp):
    s_idx = x[:, 0]
    r_idx = x[:, 1]
    o_idx = x[:, 2]
    d = x[:, 3]
    m = x[:, 4]
    rel_s = x[:, 6:6 + NR]
    rel_o = x[:, 6 + NR:6 + 2 * NR]
    w_rp2 = w_rp[:, :, 0]

    (s_rows, o_rows, sdf, sdp, sda, smf, smp, sma,
     odf, odp, oda, omf, omp, oma) = _sc_gather(
        s_idx, o_idx, e_emb, abs_d_frq_emb, abs_d_phi_emb, abs_d_amp_emb,
        abs_m_frq_emb, abs_m_phi_emb, abs_m_amp_emb)

    r_out, s_r, o_r = _rel_kernel(r_idx, rel_s, rel_o, r_emb, w_rp2)

    s_t, s_p, o_t, o_p = _tp_kernel(
        d, m, s_idx & 1, o_idx & 1, w_e, s_rows, sdf, sdp, sda, smf, smp, sma,
        o_rows, odf, odp, oda, omf, omp, oma)

    r3 = lambda a: a.reshape(B, 1, STT)
    return (r3(s_rows), r3(s_t), r3(s_p), s_r.reshape(B, REL, 1), r3(r_out),
            r3(o_rows), r3(o_t), r3(o_p), o_r.reshape(B, REL, 1))


# speedup vs baseline: 3.0057x; 3.0057x over previous
import functools

import jax
import jax.numpy as jnp
from jax import lax
from jax.experimental import pallas as pl
from jax.experimental.pallas import tpu as pltpu
from jax.experimental.pallas import tpu_sc as plsc

NENT = 100000
NR = 26
STT = 128
ABS = 128
REL = 128
RDIM = 128
B = 16384

_NC = 2
_NS = 16
_NW = _NC * _NS
_BPW = B // _NW
_CH = 128
_NCHUNK = _BPW // _CH

_NB = 256
_GRID = B // _NB


def _sc_gather_body(sidx_hbm, oidx_hbm, sidxh_hbm, oidxh_hbm, e_emb, dfrq,
                    dphi, damp, mfrq, mphi, mamp, out_s, out_o, out_sdf,
                    out_sdp, out_sda, out_smf, out_smp, out_sma, out_odf,
                    out_odp, out_oda, out_omf, out_omp, out_oma, idx_s, idx_o,
                    idx_sh, idx_oh, buf, sem):
    wid = lax.axis_index("s") * _NC + lax.axis_index("c")
    base = wid * _BPW
    pltpu.sync_copy(sidx_hbm.at[pl.ds(wid * _NCHUNK, _NCHUNK)], idx_s)
    pltpu.sync_copy(oidx_hbm.at[pl.ds(wid * _NCHUNK, _NCHUNK)], idx_o)
    pltpu.sync_copy(sidxh_hbm.at[pl.ds(wid * _NCHUNK, _NCHUNK)], idx_sh)
    pltpu.sync_copy(oidxh_hbm.at[pl.ds(wid * _NCHUNK, _NCHUNK)], idx_oh)

    jobs = (
        (idx_s, e_emb, out_s),
        (idx_o, e_emb, out_o),
        (idx_s, dfrq, out_sdf),
        (idx_s, dphi, out_sdp),
        (idx_sh, damp, out_sda),
        (idx_s, mfrq, out_smf),
        (idx_s, mphi, out_smp),
        (idx_sh, mamp, out_sma),
        (idx_o, dfrq, out_odf),
        (idx_o, dphi, out_odp),
        (idx_oh, damp, out_oda),
        (idx_o, mfrq, out_omf),
        (idx_o, mphi, out_omp),
        (idx_oh, mamp, out_oma),
    )
    for idx, table, out in jobs:
        for c in range(_NCHUNK):
            pltpu.async_copy(table.at[idx.at[c]], buf, sem).wait()
            pltpu.sync_copy(buf, out.at[pl.ds(base + c * _CH, _CH)])


def _sc_gather(s_idx, o_idx, e_emb, dfrq, dphi, damp, mfrq, mphi, mamp):
    f32 = jnp.float32
    outs = [jax.ShapeDtypeStruct((B, 128), f32) for _ in range(14)]
    mesh = plsc.VectorSubcoreMesh(core_axis_name="c", subcore_axis_name="s")
    fn = pl.kernel(
        _sc_gather_body,
        mesh=mesh,
        out_type=outs,
        scratch_types=[
            pltpu.VMEM((_NCHUNK, _CH), jnp.int32),
            pltpu.VMEM((_NCHUNK, _CH), jnp.int32),
            pltpu.VMEM((_NCHUNK, _CH), jnp.int32),
            pltpu.VMEM((_NCHUNK, _CH), jnp.int32),
            pltpu.VMEM((_CH, 128), f32),
            pltpu.SemaphoreType.DMA,
        ],
    )
    r2 = lambda a: a.reshape(B // _CH, _CH)
    return fn(r2(s_idx), r2(o_idx), r2(s_idx >> 1), r2(o_idx >> 1),
              e_emb, dfrq, dphi, damp.reshape(NENT // 2, 128), mfrq, mphi,
              mamp.reshape(NENT // 2, 128))


_SIN_COEF = (1.0, -0.16666666, 8.3333142e-03, -1.9840311e-04,
             2.7532292e-06, -2.4701608e-08, 1.3533263e-10)
_INV2PI = 0.15915494309189535
_C1 = 6.2831855
_C2 = -1.7484556025237907e-07


def _sin_approx(x):
    t = x * _INV2PI
    n = jnp.floor(t + 0.5)
    y = x - n * _C1
    y = y + n * _C2
    s = y * y
    p = jnp.full_like(x, _SIN_COEF[6])
    for c in (_SIN_COEF[5], _SIN_COEF[4], _SIN_COEF[3], _SIN_COEF[2],
              _SIN_COEF[1], _SIN_COEF[0]):
        p = p * s + c
    return y * p


def _rel_body(ridx_ref, rels_ref, relo_ref, remb_ref, wrp_ref,
              r_out_ref, sr_ref, or_ref):
    f32 = jnp.float32
    t = lax.broadcasted_iota(jnp.int32, (1, REL // 2), 1).astype(f32) * (2.0 / REL)
    frq = jnp.exp(t * (-jnp.log(10000.0)))
    frq2 = jnp.concatenate([frq, frq], axis=1)
    lane = lax.broadcasted_iota(jnp.int32, (1, REL), 1)
    phase = jnp.where(lane < REL // 2, f32(1.5707964), f32(0.0))

    ridx = ridx_ref[...]
    iota_r = lax.broadcasted_iota(jnp.int32, (1, NR), 1)
    oh = (ridx == iota_r).astype(f32)
    r_out_ref[...] = jnp.dot(oh, remb_ref[...], preferred_element_type=f32)
    w_sel = jnp.dot(oh, wrp_ref[...], preferred_element_type=f32)

    for rel_ref, out_ref in ((rels_ref, sr_ref), (relo_ref, or_ref)):
        e = rel_ref[...].astype(f32)
        acc = jnp.zeros((_NB, REL), f32)
        for j in range(NR):
            x = e[:, j:j + 1] * frq2 + phase
            acc = acc + w_sel[:, j:j + 1] * _sin_approx(x)
        out_ref[...] = acc


def _rel_kernel(r_idx, rel_s, rel_o, r_emb, w_rp2):
    f32 = jnp.float32
    blk = lambda w: pl.BlockSpec((_NB, w), lambda i: (i, 0))
    whole = lambda a, b: pl.BlockSpec((a, b), lambda i: (0, 0))
    return pl.pallas_call(
        _rel_body,
        grid=(_GRID,),
        in_specs=[blk(1), blk(NR), blk(NR), whole(NR, REL), whole(NR, NR)],
        out_specs=[blk(REL), blk(REL), blk(REL)],
        out_shape=[jax.ShapeDtypeStruct((B, REL), f32)] * 3,
    )(r_idx.reshape(B, 1), rel_s, rel_o, r_emb, w_rp2)


def _tp_body(d_ref, m_ref, spar_ref, opar_ref, we_ref, s_ref, sdf_ref,
             sdp_ref, sda_ref, smf_ref, smp_ref, sma_ref, o_ref, odf_ref,
             odp_ref, oda_ref, omf_ref, omp_ref, oma_ref,
             st_ref, sp_ref, ot_ref, op_ref):
    f32 = jnp.float32
    H = ABS // 2
    d = d_ref[...].astype(f32)
    m = m_ref[...].astype(f32)
    spar = spar_ref[...] > 0
    opar = opar_ref[...] > 0

    we = we_ref[...]
    re_w = we[:H, :]
    im_w = we[H:, :]
    w_top = jnp.concatenate([re_w, -im_w], axis=1)
    w_bot = jnp.concatenate([-im_w, re_w], axis=1)
    w_full = jnp.concatenate([w_top, w_bot], axis=0)

    lane = lax.broadcasted_iota(jnp.int32, (1, STT), 1)
    phase = jnp.where(lane < H, f32(1.5707964), f32(0.0))

    def t_half(scale, frq_ref, phi_ref, amp_ref, par):
        a = scale * frq_ref[...] + phi_ref[...] + phase
        ampf = amp_ref[...]
        amp = jnp.where(par, ampf[:, H:], ampf[:, :H])
        return jnp.concatenate([amp, amp], axis=1) * _sin_approx(a)

    st_ref[...] = (t_half(d, sdf_ref, sdp_ref, sda_ref, spar) +
                   t_half(m, smf_ref, smp_ref, sma_ref, spar))
    ot_ref[...] = (t_half(d, odf_ref, odp_ref, oda_ref, opar) +
                   t_half(m, omf_ref, omp_ref, oma_ref, opar))
    sp_ref[...] = jnp.dot(s_ref[...], w_full, preferred_element_type=f32)
    op_ref[...] = jnp.dot(o_ref[...], w_full, preferred_element_type=f32)


def _tp_kernel(d, m, s_par, o_par, w_e, s_rows, sdf, sdp, sda, smf, smp, sma,
               o_rows, odf, odp, oda, omf, omp, oma):
    f32 = jnp.float32
    blk = lambda w: pl.BlockSpec((_NB, w), lambda i: (i, 0))
    side = [blk(ABS)] * 6
    return pl.pallas_call(
        _tp_body,
        grid=(_GRID,),
        in_specs=([blk(1), blk(1), blk(1), blk(1),
                   pl.BlockSpec((STT, STT // 2), lambda i: (0, 0)),
                   blk(STT)] + side + [blk(STT)] + side),
        out_specs=[blk(STT)] * 4,
        out_shape=[jax.ShapeDtypeStruct((B, STT), f32)] * 4,
    )(d.reshape(B, 1), m.reshape(B, 1), s_par.reshape(B, 1),
      o_par.reshape(B, 1), w_e, s_rows, sdf, sdp, sda, smf, smp,
      sma, o_rows, odf, odp, oda, omf, omp, oma)


def kernel(x, e_emb, r_emb, abs_d_frq_emb, abs_d_phi_emb, abs_d_amp_emb,
           abs_m_frq_emb, abs_m_phi_emb, abs_m_amp_emb, w_e, w_rp):
    s_idx = x[:, 0]
    r_idx = x[:, 1]
    o_idx = x[:, 2]
    d = x[:, 3]
    m = x[:, 4]
    rel_s = x[:, 6:6 + NR]
    rel_o = x[:, 6 + NR:6 + 2 * NR]
    w_rp2 = w_rp[:, :, 0]

    (s_rows, o_rows, sdf, sdp, sda, smf, smp, sma,
     odf, odp, oda, omf, omp, oma) = _sc_gather(
        s_idx, o_idx, e_emb, abs_d_frq_emb, abs_d_phi_emb, abs_d_amp_emb,
        abs_m_frq_emb, abs_m_phi_emb, abs_m_amp_emb)

    r_out, s_r, o_r = _rel_kernel(r_idx, rel_s, rel_o, r_emb, w_rp2)

    s_t, s_p, o_t, o_p = _tp_kernel(
        d, m, s_idx & 1, o_idx & 1, w_e, s_rows, sdf, sdp, sda, smf, smp, sma,
        o_rows, odf, odp, oda, omf, omp, oma)

    r3 = lambda a: a.reshape(B, 1, STT)
    return (r3(s_rows), r3(s_t), r3(s_p), s_r.reshape(B, REL, 1), r3(r_out),
            r3(o_rows), r3(o_t), r3(o_p), o_r.reshape(B, REL, 1))
